# Initial kernel scaffold; baseline (speedup 1.0000x reference)
import jax
import jax.numpy as jnp
from jax.experimental import pallas as pl


def kernel(features, center_points, boundary_points):
    raise NotImplementedError("write your pallas kernel here")


# speedup vs baseline: 7.7863x; 7.7863x over previous
import functools

import jax
import jax.numpy as jnp
from jax import lax
from jax.experimental import pallas as pl
from jax.experimental.pallas import tpu as pltpu
from jax.experimental.pallas import tpu_sc as plsc

NC = 2
NS = 16
LANES = 16

C = 192
H = 64
W = 64
NROI = 32
WG = 32
HG = 5
NPTS = NROI * HG * WG
CPW = C // NS


def _sc_body(feat_hbm, bxu, byu, bxd, byd, cxh, cyh, out_hbm,
             bxu_v, byu_v, bxd_v, byd_v, cx_v, cy_v,
             idxa_v, idxb_v, idxc_v, idxd_v,
             wa_v, wb_v, wc_v, wd_v,
             p0_v, p1_v, ps_v, out_v):
    b = lax.axis_index("c")
    s = lax.axis_index("s")
    ch0 = s * CPW

    pltpu.sync_copy(bxu.at[b], bxu_v)
    pltpu.sync_copy(byu.at[b], byu_v)
    pltpu.sync_copy(bxd.at[b], bxd_v)
    pltpu.sync_copy(byd.at[b], byd_v)
    pltpu.sync_copy(cxh.at[b], cx_v)
    pltpu.sync_copy(cyh.at[b], cy_v)

    def axis_terms(g, hi):
        p = (g + 1.0) * (0.5 * (hi - 1.0))
        pi = p.astype(jnp.int32).astype(jnp.float32)
        p0f = jnp.where(pi > p, pi - 1.0, pi)
        frac = p - p0f
        p1f = p0f + 1.0
        v0 = jnp.where((p0f >= 0.0) & (p0f <= hi - 1.0), 1.0, 0.0)
        v1 = jnp.where((p1f >= 0.0) & (p1f <= hi - 1.0), 1.0, 0.0)
        c0 = jnp.clip(p0f, 0.0, hi - 1.0).astype(jnp.int32)
        c1 = jnp.clip(p1f, 0.0, hi - 1.0).astype(jnp.int32)
        return c0, c1, frac, v0, v1

    def pts_body(n, carry):
        for wc in range(2):
            col = pl.ds(wc * LANES, LANES)
            ux = bxu_v[n, col]
            uy = byu_v[n, col]
            dx = bxd_v[n, col]
            dy = byd_v[n, col]
            for h in range(HG):
                if h == HG // 2:
                    gx = cx_v[n, col]
                    gy = cy_v[n, col]
                else:
                    t = h / (HG - 1.0)
                    gx = ux + (dx - ux) * t
                    gy = uy + (dy - uy) * t
                x0, x1, fx, vx0, vx1 = axis_terms(gx, W)
                y0, y1, fy, vy0, vy1 = axis_terms(gy, H)
                ofx = 1.0 - fx
                ofy = 1.0 - fy
                base = n * (HG * WG) + h * WG + wc * LANES
                sl = pl.ds(base, LANES)
                idxa_v[sl] = y0 * W + x0
                idxb_v[sl] = y1 * W + x0
                idxc_v[sl] = y0 * W + x1
                idxd_v[sl] = y1 * W + x1
                wa_v[sl] = ofx * ofy * (vx0 * vy0)
                wb_v[sl] = ofx * fy * (vx0 * vy1)
                wc_v[sl] = fx * ofy * (vx1 * vy0)
                wd_v[sl] = fx * fy * (vx1 * vy1)
        return carry

    lax.fori_loop(0, NROI, pts_body, 0)

    def chan_body(ci, carry):
        ch = ch0 + ci
        row0 = b * C + ch
        pltpu.sync_copy(feat_hbm.at[row0], p0_v)
        pltpu.sync_copy(feat_hbm.at[row0 + NC * C], p1_v)

        def add_body(i, c2):
            sl = pl.ds(i * LANES, LANES)
            ps_v[sl] = p0_v[sl] + p1_v[sl]
            return c2

        lax.fori_loop(0, (H * W) // LANES, add_body, 0)

        def g_body(n, c2):
            for j in range(HG * WG // LANES):
                sl = pl.ds(n * (HG * WG) + j * LANES, LANES)
                va = plsc.load_gather(ps_v, [idxa_v[sl]])
                vb = plsc.load_gather(ps_v, [idxb_v[sl]])
                vc = plsc.load_gather(ps_v, [idxc_v[sl]])
                vd = plsc.load_gather(ps_v, [idxd_v[sl]])
                acc = (va * wa_v[sl] + vb * wb_v[sl]
                       + vc * wc_v[sl] + vd * wd_v[sl])
                out_v[n, pl.ds(j * LANES, LANES)] = acc
            return c2

        lax.fori_loop(0, NROI, g_body, 0)
        pltpu.sync_copy(out_v, out_hbm.at[b, :, ch, :])
        return carry

    lax.fori_loop(0, CPW, chan_body, 0)


@jax.jit
def kernel(features, center_points, boundary_points):
    L, B, _, _, _ = features.shape
    feat2d = features.reshape(L * B * C, H * W)
    bxu = boundary_points[..., 0]
    byu = boundary_points[..., 1]
    bxd = boundary_points[..., 2]
    byd = boundary_points[..., 3]
    cx = center_points[..., 0]
    cy = center_points[..., 1]

    mesh = plsc.VectorSubcoreMesh(core_axis_name="c", subcore_axis_name="s",
                                  num_cores=NC, num_subcores=NS)
    f32 = jnp.float32
    run = pl.kernel(
        _sc_body,
        compiler_params=pltpu.CompilerParams(needs_layout_passes=False),
        out_type=jax.ShapeDtypeStruct((B, NROI, C, HG * WG), f32),
        mesh=mesh,
        scratch_types=[
            pltpu.VMEM((NROI, WG), f32),
            pltpu.VMEM((NROI, WG), f32),
            pltpu.VMEM((NROI, WG), f32),
            pltpu.VMEM((NROI, WG), f32),
            pltpu.VMEM((NROI, WG), f32),
            pltpu.VMEM((NROI, WG), f32),
            pltpu.VMEM((NPTS,), jnp.int32),
            pltpu.VMEM((NPTS,), jnp.int32),
            pltpu.VMEM((NPTS,), jnp.int32),
            pltpu.VMEM((NPTS,), jnp.int32),
            pltpu.VMEM((NPTS,), f32),
            pltpu.VMEM((NPTS,), f32),
            pltpu.VMEM((NPTS,), f32),
            pltpu.VMEM((NPTS,), f32),
            pltpu.VMEM((H * W,), f32),
            pltpu.VMEM((H * W,), f32),
            pltpu.VMEM((H * W,), f32),
            pltpu.VMEM((NROI, HG * WG), f32),
        ],
    )
    out = run(feat2d, bxu, byu, bxd, byd, cx, cy)
    out = out.reshape(B, NROI, C, HG, WG)
    return tuple(out[i] for i in range(B))
